# baseline (speedup 1.0000x reference)
import functools

import jax
import jax.numpy as jnp
from jax import lax
from jax.experimental import pallas as pl
from jax.experimental.pallas import tpu as pltpu
from jax.experimental.pallas import tpu_sc as plsc

N = 16384
D = 64
M = 1_000_000
NC = 2
NS = 16
L = 16
NW = NC * NS
R = 31264
R16 = R // L
TR = NW * R16
NB = N // L
BPW = N // NW
KB = BPW // L

_mesh = plsc.VectorSubcoreMesh(core_axis_name="c", subcore_axis_name="s")
_params = pltpu.CompilerParams(needs_layout_passes=False, use_tc_tiling_on_sc=False)


def _wid():
    return lax.axis_index("s") * NC + lax.axis_index("c")


@functools.partial(
    pl.kernel,
    out_type=jax.ShapeDtypeStruct((TR, L), jnp.int32),
    mesh=_mesh,
    scratch_types=[
        pltpu.VMEM((NB, L), jnp.int32),
        pltpu.VMEM((R16, L), jnp.int32),
    ],
    compiler_params=_params,
)
def _winner(idx2_hbm, t_hbm, idx_v, t_v):
    wid = _wid()
    base = wid * R
    pltpu.sync_copy(idx2_hbm, idx_v)

    def body(k, carry):
        e = idx_v[k, :]
        local = e - base
        mask = plsc.bitcast(local, jnp.uint32) < jnp.uint32(R)
        jv = k * L + lax.iota(jnp.int32, L)
        plsc.store_scatter(t_v, [local >> 4, local & 15], jv, mask=mask)
        return carry

    lax.fori_loop(0, NB, body, 0, unroll=8)
    pltpu.sync_copy(t_v, t_hbm.at[pl.ds(wid * R16, R16)])


@functools.partial(
    pl.kernel,
    out_type=jax.ShapeDtypeStruct((N, D), jnp.float32),
    mesh=_mesh,
    scratch_types=[
        pltpu.VMEM((BPW,), jnp.int32),
        pltpu.VMEM((BPW,), jnp.int32),
        pltpu.VMEM((BPW, L), jnp.int32),
        pltpu.VMEM((BPW,), jnp.int32),
        pltpu.VMEM((BPW, D), jnp.float32),
        pltpu.SemaphoreType.DMA,
        pltpu.SemaphoreType.DMA,
    ],
    compiler_params=_params,
)
def _route(idx_hbm, t_hbm, val_hbm, out_hbm,
           idxb_v, trow_v, trows_v, w_v, rows_v, sem1, sem2):
    wid = _wid()
    base = wid * BPW
    pltpu.sync_copy(idx_hbm.at[pl.ds(base, BPW)], idxb_v)

    def rows_body(k, carry):
        trow_v[pl.ds(k * L, L)] = idxb_v[pl.ds(k * L, L)] >> 4
        return carry

    lax.fori_loop(0, KB, rows_body, 0, unroll=8)
    pltpu.async_copy(t_hbm.at[trow_v], trows_v, sem1).wait()

    lanes = lax.iota(jnp.int32, L)

    def extract_body(k, carry):
        col = idxb_v[pl.ds(k * L, L)] & 15
        wv = plsc.load_gather(trows_v, [k * L + lanes, col])
        w_v[pl.ds(k * L, L)] = wv
        return carry

    lax.fori_loop(0, KB, extract_body, 0, unroll=8)
    pltpu.async_copy(val_hbm.at[w_v], rows_v, sem2).wait()
    pltpu.sync_copy(rows_v, out_hbm.at[pl.ds(base, BPW)])


def kernel(lift_fresh_copy_1, index_put_1, view):
    del index_put_1
    idx = lift_fresh_copy_1.astype(jnp.int32)
    t = _winner(idx.reshape(NB, L))
    return _route(idx, t, view)


# speedup vs baseline: 1.0865x; 1.0865x over previous
import functools

import jax
import jax.numpy as jnp
from jax import lax
from jax.experimental import pallas as pl
from jax.experimental.pallas import tpu as pltpu
from jax.experimental.pallas import tpu_sc as plsc

N = 16384
D = 64
M = 1_000_000
NC = 2
NS = 16
L = 16
NW = NC * NS
R = 31264
R16 = R // L
TR = NW * R16
NB = N // L
BPW = N // NW
KB = BPW // L

_mesh = plsc.VectorSubcoreMesh(core_axis_name="c", subcore_axis_name="s")
_params = pltpu.CompilerParams(needs_layout_passes=False, use_tc_tiling_on_sc=False)


def _wid():
    return lax.axis_index("s") * NC + lax.axis_index("c")


@functools.partial(
    pl.kernel,
    out_type=jax.ShapeDtypeStruct((TR, L), jnp.int32),
    mesh=_mesh,
    scratch_types=[
        pltpu.VMEM((NB, L), jnp.int32),
        pltpu.VMEM((R16, L), jnp.int32),
    ],
    compiler_params=_params,
)
def _winner(idx2_hbm, t_hbm, idx_v, t_v):
    wid = _wid()
    base = wid * R
    pltpu.sync_copy(idx2_hbm, idx_v)

    lanes = lax.iota(jnp.int32, L)
    U = 8

    def body(k8, carry):
        k0 = k8 * U
        es = [idx_v[k0 + u, :] for u in range(U)]
        for u in range(U):
            local = es[u] - base
            mask = plsc.bitcast(local, jnp.uint32) < jnp.uint32(R)
            jv = (k0 + u) * L + lanes
            plsc.store_scatter(t_v, [local >> 4, local & 15], jv, mask=mask)
        return carry

    lax.fori_loop(0, NB // U, body, 0)
    pltpu.sync_copy(t_v, t_hbm.at[pl.ds(wid * R16, R16)])


@functools.partial(
    pl.kernel,
    out_type=jax.ShapeDtypeStruct((N, D), jnp.float32),
    mesh=_mesh,
    scratch_types=[
        pltpu.VMEM((BPW,), jnp.int32),
        pltpu.VMEM((BPW,), jnp.int32),
        pltpu.VMEM((BPW, L), jnp.int32),
        pltpu.VMEM((BPW,), jnp.int32),
        pltpu.VMEM((BPW, D), jnp.float32),
        pltpu.SemaphoreType.DMA,
        pltpu.SemaphoreType.DMA,
    ],
    compiler_params=_params,
)
def _route(idx_hbm, t_hbm, val_hbm, out_hbm,
           idxb_v, trow_v, trows_v, w_v, rows_v, sem1, sem2):
    wid = _wid()
    base = wid * BPW
    pltpu.sync_copy(idx_hbm.at[pl.ds(base, BPW)], idxb_v)

    def rows_body(k, carry):
        trow_v[pl.ds(k * L, L)] = idxb_v[pl.ds(k * L, L)] >> 4
        return carry

    lax.fori_loop(0, KB, rows_body, 0, unroll=8)
    pltpu.async_copy(t_hbm.at[trow_v], trows_v, sem1).wait()

    lanes = lax.iota(jnp.int32, L)

    def extract_body(k, carry):
        col = idxb_v[pl.ds(k * L, L)] & 15
        wv = plsc.load_gather(trows_v, [k * L + lanes, col])
        w_v[pl.ds(k * L, L)] = wv
        return carry

    lax.fori_loop(0, KB, extract_body, 0, unroll=8)
    pltpu.async_copy(val_hbm.at[w_v], rows_v, sem2).wait()
    pltpu.sync_copy(rows_v, out_hbm.at[pl.ds(base, BPW)])


def kernel(lift_fresh_copy_1, index_put_1, view):
    del index_put_1
    idx = lift_fresh_copy_1.astype(jnp.int32)
    t = _winner(idx.reshape(NB, L))
    return _route(idx, t, view)
